# Initial kernel scaffold; baseline (speedup 1.0000x reference)
import functools

import jax
import jax.numpy as jnp
from jax import lax
from jax.experimental import pallas as pl
from jax.experimental.pallas import tpu as pltpu
from jax.experimental.pallas import tpu_sc as plsc

N_NODES = 10000
IN_CH = 128
HID = 64
OUT_CH = 128
N_EDGES = 320000

NC = 2
NS = 16
NW = NC * NS
C = 128
K = -(-N_EDGES // (NW * C))
E_PAD = NW * K * C
N_PAD = 10240
ZR = N_PAD // NS
DUMMY_DST = N_NODES

BLK = 1280
GRID = N_PAD // BLK

_mesh = plsc.VectorSubcoreMesh(
    core_axis_name="c", subcore_axis_name="s", num_cores=NC, num_subcores=NS)


@functools.partial(
    pl.kernel,
    out_type=jax.ShapeDtypeStruct((NC, N_PAD, 16), jnp.float32),
    mesh=_mesh,
    scratch_types=[
        pltpu.VMEM((K, C), jnp.int32),
        pltpu.VMEM((C, 16), jnp.float32),
        pltpu.VMEM((C, 16), jnp.float32),
        pltpu.VMEM_SHARED((N_PAD, 16), jnp.float32),
    ],
)
def _sc_degree(dst_hbm, out_hbm, dst_v, ones_v, zbuf, acc):
    cid = lax.axis_index("c")
    sid = lax.axis_index("s")
    wid = sid * NC + cid

    def fill(i, _):
        ones_v[i, :] = jnp.ones((16,), jnp.float32)
        zbuf[i, :] = jnp.zeros((16,), jnp.float32)
        return 0
    lax.fori_loop(0, C, fill, 0)

    for r in range(ZR // C):
        pltpu.sync_copy(zbuf, acc.at[pl.ds(sid * ZR + r * C, C)])

    pltpu.sync_copy(dst_hbm.at[wid], dst_v)
    plsc.subcore_barrier()

    def chunk(j, _):
        pltpu.sync_copy(ones_v, acc.at[dst_v.at[j]], add=True)
        return 0
    lax.fori_loop(0, K, chunk, 0)

    plsc.subcore_barrier()
    pltpu.sync_copy(acc.at[pl.ds(sid * ZR, ZR)],
                    out_hbm.at[cid, pl.ds(sid * ZR, ZR)])


@functools.partial(
    pl.kernel,
    out_type=jax.ShapeDtypeStruct((NC, N_PAD, HID), jnp.float32),
    mesh=_mesh,
    scratch_types=[
        pltpu.VMEM((K, C), jnp.int32),
        pltpu.VMEM((K, C), jnp.int32),
        pltpu.VMEM((C, HID), jnp.float32),
        pltpu.VMEM((C, HID), jnp.float32),
        pltpu.VMEM_SHARED((N_PAD, HID), jnp.float32),
        pltpu.SemaphoreType.DMA,
    ],
)
def _sc_aggregate(src_hbm, dst_hbm, g_hbm, out_hbm,
                  src_v, dst_v, rows, zbuf, acc, sem):
    cid = lax.axis_index("c")
    sid = lax.axis_index("s")
    wid = sid * NC + cid

    def z(i, _):
        for t in range(HID // 16):
            zbuf[i, pl.ds(16 * t, 16)] = jnp.zeros((16,), jnp.float32)
        return 0
    lax.fori_loop(0, C, z, 0)
    for r in range(ZR // C):
        pltpu.sync_copy(zbuf, acc.at[pl.ds(sid * ZR + r * C, C)])

    pltpu.sync_copy(src_hbm.at[wid], src_v)
    pltpu.sync_copy(dst_hbm.at[wid], dst_v)
    plsc.subcore_barrier()

    def chunk(j, _):
        pltpu.async_copy(g_hbm.at[src_v.at[j]], rows, sem).wait()
        pltpu.sync_copy(rows, acc.at[dst_v.at[j]], add=True)
        return 0
    lax.fori_loop(0, K, chunk, 0)

    plsc.subcore_barrier()
    pltpu.sync_copy(acc.at[pl.ds(sid * ZR, ZR)],
                    out_hbm.at[cid, pl.ds(sid * ZR, ZR)])


def _dinv_of(degp_blk):
    deg = degp_blk[0, :, 0:1] + degp_blk[1, :, 0:1] + 1.0
    return lax.rsqrt(deg)


def _tc_prep_body(degp, x, w1, g1):
    dinv = _dinv_of(degp[...])
    h = jnp.dot(x[...], w1[...], preferred_element_type=jnp.float32)
    g1[...] = dinv * h


def _tc_mid_body(degp, p, g, b, w2, g_next):
    dinv = _dinv_of(degp[...])
    s = p[0] + p[1] + g[...]
    r = jnp.maximum(dinv * s + b[...], 0.0)
    g_next[...] = dinv * jnp.dot(r, w2[...], preferred_element_type=jnp.float32)


def _tc_mid2_body(degp, p, g, b, g_next):
    dinv = _dinv_of(degp[...])
    s = p[0] + p[1] + g[...]
    g_next[...] = dinv * jnp.maximum(dinv * s + b[...], 0.0)


def _tc_final_body(degp, p, g, b, w3, out):
    dinv = _dinv_of(degp[...])
    a = dinv * (p[0] + p[1] + g[...])
    out[...] = jnp.dot(a, w3[...], preferred_element_type=jnp.float32) + b[...]


def _degp_spec():
    return pl.BlockSpec((NC, BLK, 16), lambda i: (0, i, 0))


def _p_spec():
    return pl.BlockSpec((NC, BLK, HID), lambda i: (0, i, 0))


def _rows_spec(d):
    return pl.BlockSpec((BLK, d), lambda i: (i, 0))


def _full_spec(shape):
    return pl.BlockSpec(shape, lambda i: tuple(0 for _ in shape))


def _tc_prep(degp, x, w1):
    return pl.pallas_call(
        _tc_prep_body,
        grid=(GRID,),
        in_specs=[_degp_spec(), _rows_spec(IN_CH), _full_spec((IN_CH, HID))],
        out_specs=_rows_spec(HID),
        out_shape=jax.ShapeDtypeStruct((N_PAD, HID), jnp.float32),
    )(degp, x, w1)


def _tc_mid(degp, p, g, b, w2):
    return pl.pallas_call(
        _tc_mid_body,
        grid=(GRID,),
        in_specs=[_degp_spec(), _p_spec(), _rows_spec(HID),
                  _full_spec((1, HID)), _full_spec((HID, HID))],
        out_specs=_rows_spec(HID),
        out_shape=jax.ShapeDtypeStruct((N_PAD, HID), jnp.float32),
    )(degp, p, g, b, w2)


def _tc_mid2(degp, p, g, b):
    return pl.pallas_call(
        _tc_mid2_body,
        grid=(GRID,),
        in_specs=[_degp_spec(), _p_spec(), _rows_spec(HID),
                  _full_spec((1, HID))],
        out_specs=_rows_spec(HID),
        out_shape=jax.ShapeDtypeStruct((N_PAD, HID), jnp.float32),
    )(degp, p, g, b)


def _tc_final(degp, p, g, b, w3):
    return pl.pallas_call(
        _tc_final_body,
        grid=(GRID,),
        in_specs=[_degp_spec(), _p_spec(), _rows_spec(HID),
                  _full_spec((1, OUT_CH)), _full_spec((HID, OUT_CH))],
        out_specs=_rows_spec(OUT_CH),
        out_shape=jax.ShapeDtypeStruct((N_PAD, OUT_CH), jnp.float32),
    )(degp, p, g, b, w3)


def kernel(x, edge_index, W1, b1, W2, b2, W3, b3):
    src = edge_index[0].astype(jnp.int32)
    dst = edge_index[1].astype(jnp.int32)
    pad = E_PAD - N_EDGES
    src_p = jnp.concatenate([src, jnp.zeros((pad,), jnp.int32)])
    dst_p = jnp.concatenate([dst, jnp.full((pad,), DUMMY_DST, jnp.int32)])
    src_p = src_p.reshape(NW, K, C)
    dst_p = dst_p.reshape(NW, K, C)

    x_p = jnp.pad(x, ((0, N_PAD - N_NODES), (0, 0)))
    b1r = b1.reshape(1, HID)
    b2r = b2.reshape(1, HID)
    b3r = b3.reshape(1, OUT_CH)

    degp = _sc_degree(dst_p)

    g1 = _tc_prep(degp, x_p, W1)
    p1 = _sc_aggregate(src_p, dst_p, g1)
    g2 = _tc_mid(degp, p1, g1, b1r, W2)
    p2 = _sc_aggregate(src_p, dst_p, g2)
    g3 = _tc_mid2(degp, p2, g2, b2r)
    p3 = _sc_aggregate(src_p, dst_p, g3)
    out = _tc_final(degp, p3, g3, b3r, W3)
    return out[:N_NODES]


# speedup vs baseline: 9.3403x; 9.3403x over previous
import functools

import jax
import jax.numpy as jnp
from jax import lax
from jax.experimental import pallas as pl
from jax.experimental.pallas import tpu as pltpu
from jax.experimental.pallas import tpu_sc as plsc

N_NODES = 10000
IN_CH = 128
HID = 64
OUT_CH = 128
N_EDGES = 320000

D = 128

NC = 2
NS = 16
NW = NC * NS
C = 128
K = -(-N_EDGES // (NW * C))
E_PAD = NW * K * C
N_PAD = 10240
ZR = N_PAD // NS
DUMMY_DST = N_NODES

BLK = 1280
GRID = N_PAD // BLK

_mesh = plsc.VectorSubcoreMesh(
    core_axis_name="c", subcore_axis_name="s", num_cores=NC, num_subcores=NS)


@functools.partial(
    pl.kernel,
    out_type=jax.ShapeDtypeStruct((NC, N_PAD, D), jnp.float32),
    mesh=_mesh,
    scratch_types=[
        pltpu.VMEM((C,), jnp.int32),
        pltpu.VMEM((C, D), jnp.float32),
        pltpu.VMEM((C, D), jnp.float32),
        pltpu.VMEM_SHARED((N_PAD, D), jnp.float32),
    ],
)
def _sc_degree(dst_hbm, out_hbm, dst_v, ones_v, zbuf, acc):
    cid = lax.axis_index("c")
    sid = lax.axis_index("s")
    wid = sid * NC + cid

    def fill(i, _):
        for t in range(D // 16):
            ones_v[i, pl.ds(16 * t, 16)] = jnp.ones((16,), jnp.float32)
            zbuf[i, pl.ds(16 * t, 16)] = jnp.zeros((16,), jnp.float32)
        return 0
    lax.fori_loop(0, C, fill, 0)

    for r in range(ZR // C):
        pltpu.sync_copy(zbuf, acc.at[pl.ds(sid * ZR + r * C, C)])

    plsc.subcore_barrier()

    def chunk(j, _):
        pltpu.sync_copy(dst_hbm.at[wid, j], dst_v)
        pltpu.sync_copy(ones_v, acc.at[dst_v], add=True)
        return 0
    lax.fori_loop(0, K, chunk, 0)

    plsc.subcore_barrier()
    pltpu.sync_copy(acc.at[pl.ds(sid * ZR, ZR)],
                    out_hbm.at[cid, pl.ds(sid * ZR, ZR)])


@functools.partial(
    pl.kernel,
    out_type=jax.ShapeDtypeStruct((NC, N_PAD, D), jnp.float32),
    mesh=_mesh,
    scratch_types=[
        pltpu.VMEM((C,), jnp.int32),
        pltpu.VMEM((C,), jnp.int32),
        pltpu.VMEM((C, D), jnp.float32),
        pltpu.VMEM((C, D), jnp.float32),
        pltpu.VMEM_SHARED((N_PAD, D), jnp.float32),
        pltpu.SemaphoreType.DMA,
    ],
)
def _sc_aggregate(src_hbm, dst_hbm, g_hbm, out_hbm,
                  src_v, dst_v, rows, zbuf, acc, sem):
    cid = lax.axis_index("c")
    sid = lax.axis_index("s")
    wid = sid * NC + cid

    def z(i, _):
        for t in range(D // 16):
            zbuf[i, pl.ds(16 * t, 16)] = jnp.zeros((16,), jnp.float32)
        return 0
    lax.fori_loop(0, C, z, 0)
    for r in range(ZR // C):
        pltpu.sync_copy(zbuf, acc.at[pl.ds(sid * ZR + r * C, C)])

    plsc.subcore_barrier()

    def chunk(j, _):
        pltpu.sync_copy(src_hbm.at[wid, j], src_v)
        pltpu.sync_copy(dst_hbm.at[wid, j], dst_v)
        pltpu.async_copy(g_hbm.at[src_v], rows, sem).wait()
        pltpu.sync_copy(rows, acc.at[dst_v], add=True)
        return 0
    lax.fori_loop(0, K, chunk, 0)

    plsc.subcore_barrier()
    pltpu.sync_copy(acc.at[pl.ds(sid * ZR, ZR)],
                    out_hbm.at[cid, pl.ds(sid * ZR, ZR)])


def _dinv_of(degp_blk):
    deg = degp_blk[0, :, 0:1] + degp_blk[1, :, 0:1] + 1.0
    return lax.rsqrt(deg)


def _tc_prep_body(degp, x, w1, g1):
    dinv = _dinv_of(degp[...])
    h = jnp.dot(x[...], w1[...], preferred_element_type=jnp.float32)
    g1[...] = dinv * h


def _tc_mid_body(degp, p, g, b, w2, g_next):
    dinv = _dinv_of(degp[...])
    s = p[0] + p[1] + g[...]
    r = jnp.maximum(dinv * s + b[...], 0.0)
    g_next[...] = dinv * jnp.dot(r, w2[...], preferred_element_type=jnp.float32)


def _tc_mid2_body(degp, p, g, b, g_next):
    dinv = _dinv_of(degp[...])
    s = p[0] + p[1] + g[...]
    g_next[...] = dinv * jnp.maximum(dinv * s + b[...], 0.0)


def _tc_final_body(degp, p, g, b, w3, out):
    dinv = _dinv_of(degp[...])
    a = dinv * (p[0] + p[1] + g[...])
    out[...] = jnp.dot(a, w3[...], preferred_element_type=jnp.float32) + b[...]


def _p_spec():
    return pl.BlockSpec((NC, BLK, D), lambda i: (0, i, 0))


def _rows_spec():
    return pl.BlockSpec((BLK, D), lambda i: (i, 0))


def _full_spec(shape):
    return pl.BlockSpec(shape, lambda i: tuple(0 for _ in shape))


def _tc_prep(degp, x, w1):
    return pl.pallas_call(
        _tc_prep_body,
        grid=(GRID,),
        in_specs=[_p_spec(), _rows_spec(), _full_spec((D, D))],
        out_specs=_rows_spec(),
        out_shape=jax.ShapeDtypeStruct((N_PAD, D), jnp.float32),
    )(degp, x, w1)


def _tc_mid(degp, p, g, b, w2):
    return pl.pallas_call(
        _tc_mid_body,
        grid=(GRID,),
        in_specs=[_p_spec(), _p_spec(), _rows_spec(),
                  _full_spec((1, D)), _full_spec((D, D))],
        out_specs=_rows_spec(),
        out_shape=jax.ShapeDtypeStruct((N_PAD, D), jnp.float32),
    )(degp, p, g, b, w2)


def _tc_mid2(degp, p, g, b):
    return pl.pallas_call(
        _tc_mid2_body,
        grid=(GRID,),
        in_specs=[_p_spec(), _p_spec(), _rows_spec(), _full_spec((1, D))],
        out_specs=_rows_spec(),
        out_shape=jax.ShapeDtypeStruct((N_PAD, D), jnp.float32),
    )(degp, p, g, b)


def _tc_final(degp, p, g, b, w3):
    return pl.pallas_call(
        _tc_final_body,
        grid=(GRID,),
        in_specs=[_p_spec(), _p_spec(), _rows_spec(),
                  _full_spec((1, D)), _full_spec((D, D))],
        out_specs=_rows_spec(),
        out_shape=jax.ShapeDtypeStruct((N_PAD, D), jnp.float32),
    )(degp, p, g, b, w3)


def kernel(x, edge_index, W1, b1, W2, b2, W3, b3):
    src = edge_index[0].astype(jnp.int32)
    dst = edge_index[1].astype(jnp.int32)
    pad = E_PAD - N_EDGES
    src_p = jnp.concatenate([src, jnp.zeros((pad,), jnp.int32)])
    dst_p = jnp.concatenate([dst, jnp.full((pad,), DUMMY_DST, jnp.int32)])
    src_p = src_p.reshape(NW, K, C)
    dst_p = dst_p.reshape(NW, K, C)

    x_p = jnp.pad(x, ((0, N_PAD - N_NODES), (0, 0)))
    w1 = jnp.pad(W1, ((0, 0), (0, D - HID)))
    w2 = jnp.pad(W2, ((0, D - HID), (0, D - HID)))
    w3 = jnp.pad(W3, ((0, D - HID), (0, 0)))
    b1r = jnp.pad(b1, (0, D - HID)).reshape(1, D)
    b2r = jnp.pad(b2, (0, D - HID)).reshape(1, D)
    b3r = b3.reshape(1, D)

    degp = _sc_degree(dst_p)

    g1 = _tc_prep(degp, x_p, w1)
    p1 = _sc_aggregate(src_p, dst_p, g1)
    g2 = _tc_mid(degp, p1, g1, b1r, w2)
    p2 = _sc_aggregate(src_p, dst_p, g2)
    g3 = _tc_mid2(degp, p2, g2, b2r)
    p3 = _sc_aggregate(src_p, dst_p, g3)
    out = _tc_final(degp, p3, g3, b3r, w3)
    return out[:N_NODES]


# speedup vs baseline: 12.3283x; 1.3199x over previous
import functools

import jax
import jax.numpy as jnp
from jax import lax
from jax.experimental import pallas as pl
from jax.experimental.pallas import tpu as pltpu
from jax.experimental.pallas import tpu_sc as plsc

N_NODES = 10000
IN_CH = 128
HID = 64
OUT_CH = 128
N_EDGES = 320000

D = 128

NC = 2
NS = 16
NW = NC * NS
C = 128
K = -(-N_EDGES // (NW * C))
E_PAD = NW * K * C
N_PAD = 10112
ZR = N_PAD // NS
DUMMY_DST = N_NODES

BLK = 1264
GRID = N_PAD // BLK

_mesh = plsc.VectorSubcoreMesh(
    core_axis_name="c", subcore_axis_name="s", num_cores=NC, num_subcores=NS)


DW = 16


@functools.partial(
    pl.kernel,
    out_type=jax.ShapeDtypeStruct((NC, N_PAD, DW), jnp.float32),
    mesh=_mesh,
    scratch_types=[
        pltpu.VMEM((K, C), jnp.int32),
        pltpu.VMEM((C, DW), jnp.float32),
        pltpu.VMEM((C, DW), jnp.float32),
        pltpu.VMEM_SHARED((N_PAD, DW), jnp.float32),
    ],
)
def _sc_degree(dst_hbm, out_hbm, dst_v, ones_v, zbuf, acc):
    cid = lax.axis_index("c")
    sid = lax.axis_index("s")
    wid = sid * NC + cid

    def fill(i, _):
        ones_v[i, :] = jnp.ones((DW,), jnp.float32)
        zbuf[i, :] = jnp.zeros((DW,), jnp.float32)
        return 0
    lax.fori_loop(0, C, fill, 0)

    def zero(r, _):
        pltpu.sync_copy(zbuf, acc.at[pl.ds(sid * ZR + r * C, C)])
        return 0
    lax.fori_loop(0, ZR // C, zero, 0)
    pltpu.sync_copy(zbuf.at[pl.ds(0, ZR % C)],
                    acc.at[pl.ds(sid * ZR + (ZR // C) * C, ZR % C)])

    pltpu.sync_copy(dst_hbm.at[wid], dst_v)
    plsc.subcore_barrier()

    def chunk(j, _):
        pltpu.sync_copy(ones_v, acc.at[dst_v.at[j]], add=True)
        return 0
    lax.fori_loop(0, K, chunk, 0)

    plsc.subcore_barrier()
    pltpu.sync_copy(acc.at[pl.ds(sid * ZR, ZR)],
                    out_hbm.at[cid, pl.ds(sid * ZR, ZR)])


@functools.partial(
    pl.kernel,
    out_type=jax.ShapeDtypeStruct((NC, N_PAD, D), jnp.float32),
    mesh=_mesh,
    scratch_types=[
        pltpu.VMEM((C,), jnp.int32),
        pltpu.VMEM((C,), jnp.int32),
        pltpu.VMEM((C,), jnp.int32),
        pltpu.VMEM((C,), jnp.int32),
        pltpu.VMEM((C, D), jnp.float32),
        pltpu.VMEM((C, D), jnp.float32),
        pltpu.VMEM((C, D), jnp.float32),
        pltpu.VMEM_SHARED((N_PAD, D), jnp.float32),
        pltpu.SemaphoreType.DMA,
        pltpu.SemaphoreType.DMA,
        pltpu.SemaphoreType.DMA,
        pltpu.SemaphoreType.DMA,
    ],
)
def _sc_aggregate(src_hbm, dst_hbm, g_hbm, out_hbm,
                  src_a, src_b, dst_a, dst_b, rows_a, rows_b, zbuf, acc,
                  semg_a, semg_b, semi_a, semi_b):
    cid = lax.axis_index("c")
    sid = lax.axis_index("s")
    wid = sid * NC + cid

    def z(i, _):
        for t in range(D // 16):
            zbuf[i, pl.ds(16 * t, 16)] = jnp.zeros((16,), jnp.float32)
        return 0
    lax.fori_loop(0, C, z, 0)

    def zero(r, _):
        pltpu.sync_copy(zbuf, acc.at[pl.ds(sid * ZR + r * C, C)])
        return 0
    lax.fori_loop(0, ZR // C, zero, 0)
    pltpu.sync_copy(zbuf.at[pl.ds(0, ZR % C)],
                    acc.at[pl.ds(sid * ZR + (ZR // C) * C, ZR % C)])

    plsc.subcore_barrier()

    assert K % 2 == 1

    def idx_load(k, sr, ds_, sem):
        pltpu.async_copy(src_hbm.at[wid, k, 0], sr, sem)
        pltpu.async_copy(dst_hbm.at[wid, k, 0], ds_, sem)

    def idx_wait(k, sr, ds_, sem):
        pltpu.make_async_copy(src_hbm.at[wid, k, 0], sr, sem).wait()
        pltpu.make_async_copy(dst_hbm.at[wid, k, 0], ds_, sem).wait()

    pltpu.sync_copy(src_hbm.at[wid, 0, 0], src_a)
    pltpu.sync_copy(dst_hbm.at[wid, 0, 0], dst_a)
    idx_load(1, src_b, dst_b, semi_b)
    pltpu.async_copy(g_hbm.at[src_a], rows_a, semg_a)

    def chunk(jj, _):
        j = 2 * jj
        pltpu.make_async_copy(g_hbm.at[src_a], rows_a, semg_a).wait()
        idx_wait(j + 1, src_b, dst_b, semi_b)
        pltpu.async_copy(g_hbm.at[src_b], rows_b, semg_b)
        pltpu.sync_copy(rows_a, acc.at[dst_a], add=True)
        idx_load(j + 2, src_a, dst_a, semi_a)
        pltpu.make_async_copy(g_hbm.at[src_b], rows_b, semg_b).wait()
        idx_wait(j + 2, src_a, dst_a, semi_a)
        pltpu.async_copy(g_hbm.at[src_a], rows_a, semg_a)
        pltpu.sync_copy(rows_b, acc.at[dst_b], add=True)
        idx_load(jnp.minimum(j + 3, K - 1), src_b, dst_b, semi_b)
        return 0
    lax.fori_loop(0, (K - 1) // 2, chunk, 0)
    pltpu.make_async_copy(g_hbm.at[src_a], rows_a, semg_a).wait()
    pltpu.sync_copy(rows_a, acc.at[dst_a], add=True)
    idx_wait(K - 1, src_b, dst_b, semi_b)

    plsc.subcore_barrier()
    pltpu.sync_copy(acc.at[pl.ds(sid * ZR, ZR)],
                    out_hbm.at[cid, pl.ds(sid * ZR, ZR)])


def _dinv_of(degp_blk):
    deg = degp_blk[0, :, 0:1] + degp_blk[1, :, 0:1] + 1.0
    return lax.rsqrt(deg)


def _tc_prep_body(degp, x, w1, g1):
    dinv = _dinv_of(degp[...])
    h = jnp.dot(x[...], w1[...], preferred_element_type=jnp.float32)
    g1[...] = dinv * h


def _tc_mid_body(degp, p, g, b, w2, g_next):
    dinv = _dinv_of(degp[...])
    s = p[0] + p[1] + g[...]
    r = jnp.maximum(dinv * s + b[...], 0.0)
    g_next[...] = dinv * jnp.dot(r, w2[...], preferred_element_type=jnp.float32)


def _tc_mid2_body(degp, p, g, b, g_next):
    dinv = _dinv_of(degp[...])
    s = p[0] + p[1] + g[...]
    g_next[...] = dinv * jnp.maximum(dinv * s + b[...], 0.0)


def _tc_final_body(degp, p, g, b, w3, out):
    dinv = _dinv_of(degp[...])
    a = dinv * (p[0] + p[1] + g[...])
    out[...] = jnp.dot(a, w3[...], preferred_element_type=jnp.float32) + b[...]


def _degp_spec():
    return pl.BlockSpec((NC, BLK, DW), lambda i: (0, i, 0))


def _p_spec():
    return pl.BlockSpec((NC, BLK, D), lambda i: (0, i, 0))


def _rows_spec():
    return pl.BlockSpec((BLK, D), lambda i: (i, 0))


def _full_spec(shape):
    return pl.BlockSpec(shape, lambda i: tuple(0 for _ in shape))


def _tc_prep(degp, x, w1):
    return pl.pallas_call(
        _tc_prep_body,
        grid=(GRID,),
        in_specs=[_degp_spec(), _rows_spec(), _full_spec((D, D))],
        out_specs=_rows_spec(),
        out_shape=jax.ShapeDtypeStruct((N_PAD, D), jnp.float32),
    )(degp, x, w1)


def _tc_mid(degp, p, g, b, w2):
    return pl.pallas_call(
        _tc_mid_body,
        grid=(GRID,),
        in_specs=[_degp_spec(), _p_spec(), _rows_spec(),
                  _full_spec((1, D)), _full_spec((D, D))],
        out_specs=_rows_spec(),
        out_shape=jax.ShapeDtypeStruct((N_PAD, D), jnp.float32),
    )(degp, p, g, b, w2)


def _tc_mid2(degp, p, g, b):
    return pl.pallas_call(
        _tc_mid2_body,
        grid=(GRID,),
        in_specs=[_degp_spec(), _p_spec(), _rows_spec(), _full_spec((1, D))],
        out_specs=_rows_spec(),
        out_shape=jax.ShapeDtypeStruct((N_PAD, D), jnp.float32),
    )(degp, p, g, b)


def _tc_final(degp, p, g, b, w3):
    return pl.pallas_call(
        _tc_final_body,
        grid=(GRID,),
        in_specs=[_degp_spec(), _p_spec(), _rows_spec(),
                  _full_spec((1, D)), _full_spec((D, D))],
        out_specs=_rows_spec(),
        out_shape=jax.ShapeDtypeStruct((N_PAD, D), jnp.float32),
    )(degp, p, g, b, w3)


def kernel(x, edge_index, W1, b1, W2, b2, W3, b3):
    src = edge_index[0].astype(jnp.int32)
    dst = edge_index[1].astype(jnp.int32)
    pad = E_PAD - N_EDGES
    src_p = jnp.concatenate([src, jnp.zeros((pad,), jnp.int32)])
    dst_p = jnp.concatenate([dst, jnp.full((pad,), DUMMY_DST, jnp.int32)])
    src_p = src_p.reshape(NW, K, C)
    dst_p = dst_p.reshape(NW, K, C)

    x_p = jnp.pad(x, ((0, N_PAD - N_NODES), (0, 0)))
    w1 = jnp.pad(W1, ((0, 0), (0, D - HID)))
    w2 = jnp.pad(W2, ((0, D - HID), (0, D - HID)))
    w3 = jnp.pad(W3, ((0, D - HID), (0, 0)))
    b1r = jnp.pad(b1, (0, D - HID)).reshape(1, D)
    b2r = jnp.pad(b2, (0, D - HID)).reshape(1, D)
    b3r = b3.reshape(1, D)

    src_p4 = src_p.reshape(NW, K, 1, C)
    dst_p4 = dst_p.reshape(NW, K, 1, C)

    degp = _sc_degree(dst_p)

    g1 = _tc_prep(degp, x_p, w1)
    p1 = _sc_aggregate(src_p4, dst_p4, g1)
    g2 = _tc_mid(degp, p1, g1, b1r, w2)
    p2 = _sc_aggregate(src_p4, dst_p4, g2)
    g3 = _tc_mid2(degp, p2, g2, b2r)
    p3 = _sc_aggregate(src_p4, dst_p4, g3)
    out = _tc_final(degp, p3, g3, b3r, w3)
    return out[:N_NODES]


# speedup vs baseline: 24.8506x; 2.0157x over previous
import functools

import jax
import jax.numpy as jnp
from jax import lax
from jax.experimental import pallas as pl
from jax.experimental.pallas import tpu as pltpu
from jax.experimental.pallas import tpu_sc as plsc

N_NODES = 10000
IN_CH = 128
HID = 64
OUT_CH = 128
N_EDGES = 320000

D = 128

NC = 2
NS = 16
NW = NC * NS
C = 128
K = -(-N_EDGES // (NW * C))
E_PAD = NW * K * C
N_PAD = 10112
ZR = N_PAD // NS
DUMMY_DST = N_NODES

BLK = 1264
GRID = N_PAD // BLK

_mesh = plsc.VectorSubcoreMesh(
    core_axis_name="c", subcore_axis_name="s", num_cores=NC, num_subcores=NS)


DW = 16


@functools.partial(
    pl.kernel,
    out_type=jax.ShapeDtypeStruct((NC, N_PAD, DW), jnp.float32),
    mesh=_mesh,
    scratch_types=[
        pltpu.VMEM((K, C), jnp.int32),
        pltpu.VMEM((C, DW), jnp.float32),
        pltpu.VMEM((C, DW), jnp.float32),
        pltpu.VMEM_SHARED((N_PAD, DW), jnp.float32),
    ],
)
def _sc_degree(dst_hbm, out_hbm, dst_v, ones_v, zbuf, acc):
    cid = lax.axis_index("c")
    sid = lax.axis_index("s")
    wid = sid * NC + cid

    def fill(i, _):
        ones_v[i, :] = jnp.ones((DW,), jnp.float32)
        zbuf[i, :] = jnp.zeros((DW,), jnp.float32)
        return 0
    lax.fori_loop(0, C, fill, 0)

    def zero(r, _):
        pltpu.sync_copy(zbuf, acc.at[pl.ds(sid * ZR + r * C, C)])
        return 0
    lax.fori_loop(0, ZR // C, zero, 0)
    pltpu.sync_copy(zbuf.at[pl.ds(0, ZR % C)],
                    acc.at[pl.ds(sid * ZR + (ZR // C) * C, ZR % C)])

    pltpu.sync_copy(dst_hbm.at[wid], dst_v)
    plsc.subcore_barrier()

    def chunk(j, _):
        pltpu.sync_copy(ones_v, acc.at[dst_v.at[j]], add=True)
        return 0
    lax.fori_loop(0, K, chunk, 0)

    plsc.subcore_barrier()
    pltpu.sync_copy(acc.at[pl.ds(sid * ZR, ZR)],
                    out_hbm.at[cid, pl.ds(sid * ZR, ZR)])


@functools.partial(
    pl.kernel,
    out_type=jax.ShapeDtypeStruct((NC, N_PAD, D), jnp.float32),
    mesh=_mesh,
    scratch_types=[
        pltpu.VMEM((C,), jnp.int32),
        pltpu.VMEM((C,), jnp.int32),
        pltpu.VMEM((C,), jnp.int32),
        pltpu.VMEM((C,), jnp.int32),
        pltpu.VMEM((C, D), jnp.float32),
        pltpu.VMEM((C, D), jnp.float32),
        pltpu.VMEM((C, D), jnp.float32),
        pltpu.VMEM_SHARED((N_PAD, D), jnp.float32),
        pltpu.SemaphoreType.DMA,
        pltpu.SemaphoreType.DMA,
        pltpu.SemaphoreType.DMA,
        pltpu.SemaphoreType.DMA,
    ],
)
def _sc_aggregate(src_hbm, dst_hbm, g_hbm, out_hbm,
                  src_a, src_b, dst_a, dst_b, rows_a, rows_b, zbuf, acc,
                  semg_a, semg_b, semi_a, semi_b):
    cid = lax.axis_index("c")
    sid = lax.axis_index("s")
    wid = sid * NC + cid

    def z(i, _):
        for t in range(D // 16):
            zbuf[i, pl.ds(16 * t, 16)] = jnp.zeros((16,), jnp.float32)
        return 0
    lax.fori_loop(0, C, z, 0)

    def zero(r, _):
        pltpu.sync_copy(zbuf, acc.at[pl.ds(sid * ZR + r * C, C)])
        return 0
    lax.fori_loop(0, ZR // C, zero, 0)
    pltpu.sync_copy(zbuf.at[pl.ds(0, ZR % C)],
                    acc.at[pl.ds(sid * ZR + (ZR // C) * C, ZR % C)])

    plsc.subcore_barrier()

    assert K % 2 == 1

    def idx_load(k, sr, ds_, sem):
        pltpu.async_copy(src_hbm.at[wid, k, 0], sr, sem)
        pltpu.async_copy(dst_hbm.at[wid, k, 0], ds_, sem)

    def idx_wait(k, sr, ds_, sem):
        pltpu.make_async_copy(src_hbm.at[wid, k, 0], sr, sem).wait()
        pltpu.make_async_copy(dst_hbm.at[wid, k, 0], ds_, sem).wait()

    pltpu.sync_copy(src_hbm.at[wid, 0, 0], src_a)
    pltpu.sync_copy(dst_hbm.at[wid, 0, 0], dst_a)
    idx_load(1, src_b, dst_b, semi_b)
    pltpu.async_copy(g_hbm.at[src_a], rows_a, semg_a)

    def chunk(jj, _):
        j = 2 * jj
        pltpu.make_async_copy(g_hbm.at[src_a], rows_a, semg_a).wait()
        idx_wait(j + 1, src_b, dst_b, semi_b)
        pltpu.async_copy(g_hbm.at[src_b], rows_b, semg_b)
        pltpu.sync_copy(rows_a, acc.at[dst_a], add=True)
        idx_load(j + 2, src_a, dst_a, semi_a)
        pltpu.make_async_copy(g_hbm.at[src_b], rows_b, semg_b).wait()
        idx_wait(j + 2, src_a, dst_a, semi_a)
        pltpu.async_copy(g_hbm.at[src_a], rows_a, semg_a)
        pltpu.sync_copy(rows_b, acc.at[dst_b], add=True)
        idx_load(jnp.minimum(j + 3, K - 1), src_b, dst_b, semi_b)
        return 0
    lax.fori_loop(0, (K - 1) // 2, chunk, 0)
    pltpu.make_async_copy(g_hbm.at[src_a], rows_a, semg_a).wait()
    pltpu.sync_copy(rows_a, acc.at[dst_a], add=True)
    idx_wait(K - 1, src_b, dst_b, semi_b)

    plsc.subcore_barrier()
    pltpu.sync_copy(acc.at[pl.ds(sid * ZR, ZR)],
                    out_hbm.at[cid, pl.ds(sid * ZR, ZR)])


def _dinv_of(degp_blk):
    deg = degp_blk[0, :, 0:1] + degp_blk[1, :, 0:1] + 1.0
    return lax.rsqrt(deg)


def _tc_prep_body(degp, x, w1, g1):
    dinv = _dinv_of(degp[...])
    h = jnp.dot(x[...], w1[...], preferred_element_type=jnp.float32)
    g1[...] = dinv * h


def _tc_mid_body(degp, p, g, b, w2, g_next):
    dinv = _dinv_of(degp[...])
    s = p[0] + p[1] + g[...]
    r = jnp.maximum(dinv * s + b[...], 0.0)
    g_next[...] = dinv * jnp.dot(r, w2[...], preferred_element_type=jnp.float32)


def _tc_mid2_body(degp, p, g, b, g_next):
    dinv = _dinv_of(degp[...])
    s = p[0] + p[1] + g[...]
    g_next[...] = dinv * jnp.maximum(dinv * s + b[...], 0.0)


def _tc_final_body(degp, p, g, b, w3, out):
    dinv = _dinv_of(degp[...])
    a = dinv * (p[0] + p[1] + g[...])
    out[...] = jnp.dot(a, w3[...], preferred_element_type=jnp.float32) + b[...]


def _degp_spec():
    return pl.BlockSpec((NC, BLK, DW), lambda i: (0, i, 0))


def _p_spec():
    return pl.BlockSpec((NC, BLK, D), lambda i: (0, i, 0))


def _rows_spec():
    return pl.BlockSpec((BLK, D), lambda i: (i, 0))


def _full_spec(shape):
    return pl.BlockSpec(shape, lambda i: tuple(0 for _ in shape))


def _tc_prep(degp, x, w1):
    return pl.pallas_call(
        _tc_prep_body,
        grid=(GRID,),
        in_specs=[_degp_spec(), _rows_spec(), _full_spec((D, D))],
        out_specs=_rows_spec(),
        out_shape=jax.ShapeDtypeStruct((N_PAD, D), jnp.float32),
    )(degp, x, w1)


def _tc_mid(degp, p, g, b, w2):
    return pl.pallas_call(
        _tc_mid_body,
        grid=(GRID,),
        in_specs=[_degp_spec(), _p_spec(), _rows_spec(),
                  _full_spec((1, D)), _full_spec((D, D))],
        out_specs=_rows_spec(),
        out_shape=jax.ShapeDtypeStruct((N_PAD, D), jnp.float32),
    )(degp, p, g, b, w2)


def _tc_mid2(degp, p, g, b):
    return pl.pallas_call(
        _tc_mid2_body,
        grid=(GRID,),
        in_specs=[_degp_spec(), _p_spec(), _rows_spec(), _full_spec((1, D))],
        out_specs=_rows_spec(),
        out_shape=jax.ShapeDtypeStruct((N_PAD, D), jnp.float32),
    )(degp, p, g, b)


def _tc_final(degp, p, g, b, w3):
    return pl.pallas_call(
        _tc_final_body,
        grid=(GRID,),
        in_specs=[_degp_spec(), _p_spec(), _rows_spec(),
                  _full_spec((1, D)), _full_spec((D, D))],
        out_specs=_rows_spec(),
        out_shape=jax.ShapeDtypeStruct((N_PAD, D), jnp.float32),
    )(degp, p, g, b, w3)


def kernel(x, edge_index, W1, b1, W2, b2, W3, b3):
    src = edge_index[0].astype(jnp.int32)
    dst = edge_index[1].astype(jnp.int32)
    pad = E_PAD - N_EDGES
    dummy_d = DUMMY_DST + jnp.arange(pad, dtype=jnp.int32) % (N_PAD - N_NODES)
    dummy_s = jnp.arange(pad, dtype=jnp.int32) % N_NODES
    src_p = jnp.concatenate([src, dummy_s])
    dst_p = jnp.concatenate([dst, dummy_d])
    src_p = src_p.reshape(NW, K, C)
    dst_p = dst_p.reshape(NW, K, C)

    x_p = jnp.pad(x, ((0, N_PAD - N_NODES), (0, 0)))
    w1 = jnp.pad(W1, ((0, 0), (0, D - HID)))
    w2 = jnp.pad(W2, ((0, D - HID), (0, D - HID)))
    w3 = jnp.pad(W3, ((0, D - HID), (0, 0)))
    b1r = jnp.pad(b1, (0, D - HID)).reshape(1, D)
    b2r = jnp.pad(b2, (0, D - HID)).reshape(1, D)
    b3r = b3.reshape(1, D)

    src_p4 = src_p.reshape(NW, K, 1, C)
    dst_p4 = dst_p.reshape(NW, K, 1, C)

    degp = _sc_degree(dst_p)

    g1 = _tc_prep(degp, x_p, w1)
    p1 = _sc_aggregate(src_p4, dst_p4, g1)
    g2 = _tc_mid(degp, p1, g1, b1r, w2)
    p2 = _sc_aggregate(src_p4, dst_p4, g2)
    g3 = _tc_mid2(degp, p2, g2, b2r)
    p3 = _sc_aggregate(src_p4, dst_p4, g3)
    out = _tc_final(degp, p3, g3, b3r, w3)
    return out[:N_NODES]


# speedup vs baseline: 25.1442x; 1.0118x over previous
import functools

import jax
import jax.numpy as jnp
from jax import lax
from jax.experimental import pallas as pl
from jax.experimental.pallas import tpu as pltpu
from jax.experimental.pallas import tpu_sc as plsc

N_NODES = 10000
IN_CH = 128
HID = 64
OUT_CH = 128
N_EDGES = 320000

D = 128

NC = 2
NS = 16
NW = NC * NS
C = 128
K = -(-N_EDGES // (NW * C))
E_PAD = NW * K * C
N_PAD = 10112
ZR = N_PAD // NS
DUMMY_DST = N_NODES

BLK = 1264
GRID = N_PAD // BLK

_mesh = plsc.VectorSubcoreMesh(
    core_axis_name="c", subcore_axis_name="s", num_cores=NC, num_subcores=NS)


DW = 16


@functools.partial(
    pl.kernel,
    out_type=jax.ShapeDtypeStruct((NC, N_PAD, DW), jnp.float32),
    mesh=_mesh,
    scratch_types=[
        pltpu.VMEM((K, C), jnp.int32),
        pltpu.VMEM((C, DW), jnp.float32),
        pltpu.VMEM((C, DW), jnp.float32),
        pltpu.VMEM_SHARED((N_PAD, DW), jnp.float32),
    ],
)
def _sc_degree(dst_hbm, out_hbm, dst_v, ones_v, zbuf, acc):
    cid = lax.axis_index("c")
    sid = lax.axis_index("s")
    wid = sid * NC + cid

    def fill(i, _):
        ones_v[i, :] = jnp.ones((DW,), jnp.float32)
        zbuf[i, :] = jnp.zeros((DW,), jnp.float32)
        return 0
    lax.fori_loop(0, C, fill, 0)

    def zero(r, _):
        pltpu.sync_copy(zbuf, acc.at[pl.ds(sid * ZR + r * C, C)])
        return 0
    lax.fori_loop(0, ZR // C, zero, 0)
    pltpu.sync_copy(zbuf.at[pl.ds(0, ZR % C)],
                    acc.at[pl.ds(sid * ZR + (ZR // C) * C, ZR % C)])

    pltpu.sync_copy(dst_hbm.at[wid], dst_v)
    plsc.subcore_barrier()

    def chunk(j, _):
        pltpu.sync_copy(ones_v, acc.at[dst_v.at[j]], add=True)
        return 0
    lax.fori_loop(0, K, chunk, 0)

    plsc.subcore_barrier()
    pltpu.sync_copy(acc.at[pl.ds(sid * ZR, ZR)],
                    out_hbm.at[cid, pl.ds(sid * ZR, ZR)])


@functools.partial(
    pl.kernel,
    out_type=jax.ShapeDtypeStruct((NC, N_PAD, D), jnp.float32),
    mesh=_mesh,
    scratch_types=[
        pltpu.VMEM((2, 1, C), jnp.int32),
        pltpu.VMEM((2, 1, C), jnp.int32),
        pltpu.VMEM((C, D), jnp.float32),
        pltpu.VMEM((C, D), jnp.float32),
        pltpu.VMEM((C, D), jnp.float32),
        pltpu.VMEM_SHARED((N_PAD, D), jnp.float32),
        pltpu.SemaphoreType.DMA,
        pltpu.SemaphoreType.DMA,
        pltpu.SemaphoreType.DMA,
        pltpu.SemaphoreType.DMA,
    ],
)
def _sc_aggregate(ei_hbm, g_hbm, out_hbm,
                  ei_a, ei_b, rows_a, rows_b, zbuf, acc,
                  semg_a, semg_b, semi_a, semi_b):
    cid = lax.axis_index("c")
    sid = lax.axis_index("s")
    wid = sid * NC + cid

    def z(i, _):
        for t in range(D // 16):
            zbuf[i, pl.ds(16 * t, 16)] = jnp.zeros((16,), jnp.float32)
        return 0
    lax.fori_loop(0, C, z, 0)

    def zero(r, _):
        pltpu.sync_copy(zbuf, acc.at[pl.ds(sid * ZR + r * C, C)])
        return 0
    lax.fori_loop(0, ZR // C, zero, 0)
    pltpu.sync_copy(zbuf.at[pl.ds(0, ZR % C)],
                    acc.at[pl.ds(sid * ZR + (ZR // C) * C, ZR % C)])

    plsc.subcore_barrier()

    assert K % 2 == 1

    pltpu.sync_copy(ei_hbm.at[wid, 0], ei_a)
    pltpu.async_copy(ei_hbm.at[wid, 1], ei_b, semi_b)
    pltpu.async_copy(g_hbm.at[ei_a.at[0, 0]], rows_a, semg_a)

    def chunk(jj, _):
        j = 2 * jj
        pltpu.make_async_copy(g_hbm.at[ei_a.at[0, 0]], rows_a, semg_a).wait()
        pltpu.make_async_copy(ei_hbm.at[wid, j + 1], ei_b, semi_b).wait()
        pltpu.async_copy(g_hbm.at[ei_b.at[0, 0]], rows_b, semg_b)
        pltpu.sync_copy(rows_a, acc.at[ei_a.at[1, 0]], add=True)
        pltpu.async_copy(ei_hbm.at[wid, j + 2], ei_a, semi_a)
        pltpu.make_async_copy(g_hbm.at[ei_b.at[0, 0]], rows_b, semg_b).wait()
        pltpu.make_async_copy(ei_hbm.at[wid, j + 2], ei_a, semi_a).wait()
        pltpu.async_copy(g_hbm.at[ei_a.at[0, 0]], rows_a, semg_a)
        pltpu.sync_copy(rows_b, acc.at[ei_b.at[1, 0]], add=True)
        pltpu.async_copy(ei_hbm.at[wid, jnp.minimum(j + 3, K - 1)], ei_b, semi_b)
        return 0
    lax.fori_loop(0, (K - 1) // 2, chunk, 0)
    pltpu.make_async_copy(g_hbm.at[ei_a.at[0, 0]], rows_a, semg_a).wait()
    pltpu.sync_copy(rows_a, acc.at[ei_a.at[1, 0]], add=True)
    pltpu.make_async_copy(ei_hbm.at[wid, K - 1], ei_b, semi_b).wait()

    plsc.subcore_barrier()
    pltpu.sync_copy(acc.at[pl.ds(sid * ZR, ZR)],
                    out_hbm.at[cid, pl.ds(sid * ZR, ZR)])


def _dinv_of(degp_blk):
    deg = degp_blk[0, :, 0:1] + degp_blk[1, :, 0:1] + 1.0
    return lax.rsqrt(deg)


def _tc_prep_body(degp, x, w1, g1):
    dinv = _dinv_of(degp[...])
    h = jnp.dot(x[...], w1[...], preferred_element_type=jnp.float32)
    g1[...] = dinv * h


def _tc_mid_body(degp, p, g, b, w2, g_next):
    dinv = _dinv_of(degp[...])
    s = p[0] + p[1] + g[...]
    r = jnp.maximum(dinv * s + b[...], 0.0)
    g_next[...] = dinv * jnp.dot(r, w2[...], preferred_element_type=jnp.float32)


def _tc_mid2_body(degp, p, g, b, g_next):
    dinv = _dinv_of(degp[...])
    s = p[0] + p[1] + g[...]
    g_next[...] = dinv * jnp.maximum(dinv * s + b[...], 0.0)


def _tc_final_body(degp, p, g, b, w3, out):
    dinv = _dinv_of(degp[...])
    a = dinv * (p[0] + p[1] + g[...])
    out[...] = jnp.dot(a, w3[...], preferred_element_type=jnp.float32) + b[...]


def _degp_spec():
    return pl.BlockSpec((NC, BLK, DW), lambda i: (0, i, 0))


def _p_spec():
    return pl.BlockSpec((NC, BLK, D), lambda i: (0, i, 0))


def _rows_spec():
    return pl.BlockSpec((BLK, D), lambda i: (i, 0))


def _full_spec(shape):
    return pl.BlockSpec(shape, lambda i: tuple(0 for _ in shape))


def _tc_prep(degp, x, w1):
    return pl.pallas_call(
        _tc_prep_body,
        grid=(GRID,),
        in_specs=[_degp_spec(), _rows_spec(), _full_spec((D, D))],
        out_specs=_rows_spec(),
        out_shape=jax.ShapeDtypeStruct((N_PAD, D), jnp.float32),
    )(degp, x, w1)


def _tc_mid(degp, p, g, b, w2):
    return pl.pallas_call(
        _tc_mid_body,
        grid=(GRID,),
        in_specs=[_degp_spec(), _p_spec(), _rows_spec(),
                  _full_spec((1, D)), _full_spec((D, D))],
        out_specs=_rows_spec(),
        out_shape=jax.ShapeDtypeStruct((N_PAD, D), jnp.float32),
    )(degp, p, g, b, w2)


def _tc_mid2(degp, p, g, b):
    return pl.pallas_call(
        _tc_mid2_body,
        grid=(GRID,),
        in_specs=[_degp_spec(), _p_spec(), _rows_spec(), _full_spec((1, D))],
        out_specs=_rows_spec(),
        out_shape=jax.ShapeDtypeStruct((N_PAD, D), jnp.float32),
    )(degp, p, g, b)


def _tc_final(degp, p, g, b, w3):
    return pl.pallas_call(
        _tc_final_body,
        grid=(GRID,),
        in_specs=[_degp_spec(), _p_spec(), _rows_spec(),
                  _full_spec((1, D)), _full_spec((D, D))],
        out_specs=_rows_spec(),
        out_shape=jax.ShapeDtypeStruct((N_PAD, D), jnp.float32),
    )(degp, p, g, b, w3)


def kernel(x, edge_index, W1, b1, W2, b2, W3, b3):
    src = edge_index[0].astype(jnp.int32)
    dst = edge_index[1].astype(jnp.int32)
    pad = E_PAD - N_EDGES
    dummy_d = DUMMY_DST + jnp.arange(pad, dtype=jnp.int32) % (N_PAD - N_NODES)
    dummy_s = jnp.arange(pad, dtype=jnp.int32) % N_NODES
    src_p = jnp.concatenate([src, dummy_s])
    dst_p = jnp.concatenate([dst, dummy_d])
    src_p = src_p.reshape(NW, K, C)
    dst_p = dst_p.reshape(NW, K, C)

    x_p = jnp.pad(x, ((0, N_PAD - N_NODES), (0, 0)))
    w1 = jnp.pad(W1, ((0, 0), (0, D - HID)))
    w2 = jnp.pad(W2, ((0, D - HID), (0, D - HID)))
    w3 = jnp.pad(W3, ((0, D - HID), (0, 0)))
    b1r = jnp.pad(b1, (0, D - HID)).reshape(1, D)
    b2r = jnp.pad(b2, (0, D - HID)).reshape(1, D)
    b3r = b3.reshape(1, D)

    ei_p = jnp.stack([src_p, dst_p], axis=2).reshape(NW, K, 2, 1, C)

    degp = _sc_degree(dst_p)

    g1 = _tc_prep(degp, x_p, w1)
    p1 = _sc_aggregate(ei_p, g1)
    g2 = _tc_mid(degp, p1, g1, b1r, w2)
    p2 = _sc_aggregate(ei_p, g2)
    g3 = _tc_mid2(degp, p2, g2, b2r)
    p3 = _sc_aggregate(ei_p, g3)
    out = _tc_final(degp, p3, g3, b3r, w3)
    return out[:N_NODES]
